# baseline (speedup 1.0000x reference)
import jax
import jax.numpy as jnp
from jax.experimental import pallas as pl
from jax.experimental.pallas import tpu as pltpu

PENALTY = 1.2
NS = 4


def _damp_kernel(ids_ref, logits_ref, out_ref, occ_ref):
    SH = logits_ref.shape[1]
    V = logits_ref.shape[2]
    sh = pl.program_id(1)
    S = SH * pl.num_programs(1)

    ids = ids_ref[0]
    vids = jax.lax.broadcasted_iota(jnp.int32, (SH, V), 1)
    j = jax.lax.broadcasted_iota(jnp.int32, (SH, V), 0) + sh * SH
    occ_half = jnp.min(jnp.where(ids == vids, j, S), axis=0, keepdims=True)

    @pl.when(sh == 0)
    def _init():
        occ_ref[...] = occ_half

    @pl.when(sh != 0)
    def _acc():
        occ_ref[...] = jnp.minimum(occ_ref[...], occ_half)

    mask = occ_ref[...] < j
    x = logits_ref[0]
    out_ref[0] = jnp.where(mask, x * (1.0 / PENALTY), x)


@jax.jit
def kernel(logits, input_ids):
    B, S, V = logits.shape
    SH = S // NS
    ids3 = input_ids.reshape(B, S, 1)
    return pl.pallas_call(
        _damp_kernel,
        grid=(B, NS),
        in_specs=[
            pl.BlockSpec((1, SH, 1), lambda b, s: (b, s, 0)),
            pl.BlockSpec((1, SH, V), lambda b, s: (b, s, 0)),
        ],
        out_specs=pl.BlockSpec((1, SH, V), lambda b, s: (b, s, 0)),
        out_shape=jax.ShapeDtypeStruct((B, S, V), logits.dtype),
        scratch_shapes=[pltpu.VMEM((1, V), jnp.int32)],
    )(ids3, logits)


# speedup vs baseline: 1.1152x; 1.1152x over previous
import jax
import jax.numpy as jnp
from jax.experimental import pallas as pl

PENALTY = 1.2
NS = 2


def _damp_kernel(ids_ref, logits_ref, out_ref):
    S = ids_ref.shape[1]
    SH = logits_ref.shape[1]
    V = logits_ref.shape[2]
    sh = pl.program_id(1)

    ids = ids_ref[0]
    vids = jax.lax.broadcasted_iota(jnp.int32, (S, V), 1)
    j = jax.lax.broadcasted_iota(jnp.int32, (S, V), 0)
    occ = jnp.min(jnp.where(ids == vids, j, S), axis=0, keepdims=True)

    s_glob = jax.lax.broadcasted_iota(jnp.int32, (SH, V), 0) + sh * SH
    mask = occ < s_glob
    x = logits_ref[0]
    out_ref[0] = jnp.where(mask, x * (1.0 / PENALTY), x)


@jax.jit
def kernel(logits, input_ids):
    B, S, V = logits.shape
    SH = S // NS
    ids3 = input_ids.reshape(B, S, 1)
    return pl.pallas_call(
        _damp_kernel,
        grid=(B, NS),
        in_specs=[
            pl.BlockSpec((1, S, 1), lambda b, s: (b, 0, 0)),
            pl.BlockSpec((1, SH, V), lambda b, s: (b, s, 0)),
        ],
        out_specs=pl.BlockSpec((1, SH, V), lambda b, s: (b, s, 0)),
        out_shape=jax.ShapeDtypeStruct((B, S, V), logits.dtype),
    )(ids3, logits)


# speedup vs baseline: 1.3054x; 1.1705x over previous
import jax
import jax.numpy as jnp
from jax.experimental import pallas as pl

PENALTY = 1.2
BV = 50048


def _damp_kernel(ids_ref, logits_ref, out_ref):
    S = ids_ref.shape[1]
    vb = pl.program_id(1)

    ids_local = ids_ref[0] - vb * BV
    vids = jax.lax.broadcasted_iota(jnp.int32, (S, BV), 1)
    oh = jnp.where(ids_local == vids, 1.0, 0.0)

    r = jax.lax.broadcasted_iota(jnp.int32, (S, S), 0)
    c = jax.lax.broadcasted_iota(jnp.int32, (S, S), 1)
    tril = jnp.where(c < r, 1.0, 0.0)

    counts = jax.lax.dot(tril, oh, preferred_element_type=jnp.float32)

    x = logits_ref[0]
    out_ref[0] = jnp.where(counts > 0.0, x * (1.0 / PENALTY), x)


@jax.jit
def kernel(logits, input_ids):
    B, S, V = logits.shape
    ids3 = input_ids.reshape(B, S, 1)
    return pl.pallas_call(
        _damp_kernel,
        grid=(B, pl.cdiv(V, BV)),
        in_specs=[
            pl.BlockSpec((1, S, 1), lambda b, v: (b, 0, 0)),
            pl.BlockSpec((1, S, BV), lambda b, v: (b, 0, v)),
        ],
        out_specs=pl.BlockSpec((1, S, BV), lambda b, v: (b, 0, v)),
        out_shape=jax.ShapeDtypeStruct((B, S, V), logits.dtype),
    )(ids3, logits)
